# baseline (speedup 1.0000x reference)
import functools
import math

import jax
import jax.numpy as jnp
from jax import lax
from jax.experimental import pallas as pl
from jax.experimental.pallas import tpu as pltpu
from jax.experimental.pallas import tpu_sc as plsc

D_MODEL = 64
SCALE = math.sqrt(D_MODEL)
NUM_CORES = 2
NUM_SUBCORES = 16
NUM_WORKERS = NUM_CORES * NUM_SUBCORES
LANES = 16


@functools.partial(jax.jit, static_argnums=(2, 3))
def _sc_embed(xf, lut2, R, C):
    CP = (C + LANES - 1) // LANES * LANES
    r_per_w = R // NUM_WORKERS
    mesh = plsc.VectorSubcoreMesh(core_axis_name="c", subcore_axis_name="s")

    @functools.partial(
        pl.kernel,
        mesh=mesh,
        out_type=jax.ShapeDtypeStruct((R, C, D_MODEL), jnp.float32),
        scratch_types=[
            pltpu.VMEM((CP,), jnp.int32),
            pltpu.VMEM((CP,), jnp.int32),
            pltpu.VMEM((CP,), jnp.int32),
            pltpu.VMEM((CP, 128), jnp.float32),
            pltpu.VMEM((C, D_MODEL), jnp.float32),
            pltpu.SemaphoreType.DMA,
        ],
        compiler_params=pltpu.CompilerParams(needs_layout_passes=False),
    )
    def k(xf_hbm, lut_hbm, out_hbm, idx_v, blk_v, off_v, gat_v, out_v, sem):
        wid = lax.axis_index("s") * NUM_CORES + lax.axis_index("c")
        base = wid * r_per_w
        zeros = jnp.zeros((LANES,), jnp.int32)
        iota = lax.iota(jnp.int32, LANES)

        @pl.loop(0, r_per_w)
        def _(g):
            p = base + g
            idx_v[pl.ds(C // LANES * LANES, LANES)] = zeros
            pltpu.sync_copy(xf_hbm.at[pl.ds(p * C, C)], idx_v.at[pl.ds(0, C)])

            @pl.loop(0, CP // LANES)
            def _(i):
                s = pl.ds(i * LANES, LANES)
                v = idx_v[s]
                blk_v[s] = lax.shift_right_logical(v, 1)
                off_v[s] = lax.shift_left(lax.bitwise_and(v, 1), 6)

            pltpu.async_copy(lut_hbm.at[blk_v], gat_v, sem).wait()

            @pl.loop(0, C)
            def _(r):
                rsplat = jnp.full((LANES,), r, jnp.int32)
                off = plsc.load_gather(off_v, [rsplat]) + iota
                for c in range(D_MODEL // LANES):
                    v = plsc.load_gather(gat_v, [rsplat, off + (c * LANES)])
                    out_v[r, pl.ds(c * LANES, LANES)] = v * SCALE

            pltpu.sync_copy(out_v, out_hbm.at[p])

    return k(xf, lut2)


def kernel(x, lut):
    lut2 = lut.reshape(lut.shape[0] // 2, 2 * lut.shape[1])
    xf = x.reshape(-1).astype(jnp.int32)
    return _sc_embed(xf, lut2, x.shape[0], x.shape[1])


# speedup vs baseline: 1.5304x; 1.5304x over previous
import functools
import math

import jax
import jax.numpy as jnp
from jax import lax
from jax.experimental import pallas as pl
from jax.experimental.pallas import tpu as pltpu
from jax.experimental.pallas import tpu_sc as plsc

D_MODEL = 64
SCALE = math.sqrt(D_MODEL)
NUM_CORES = 2
NUM_SUBCORES = 16
NUM_WORKERS = NUM_CORES * NUM_SUBCORES
LANES = 16
RBLK = 128


@functools.partial(jax.jit, static_argnums=(2, 3))
def _sc_embed(xf, lut, R, C):
    nrb = R // RBLK
    n_blocks = C * nrb
    t_per_w = n_blocks // NUM_WORKERS
    mesh = plsc.VectorSubcoreMesh(core_axis_name="c", subcore_axis_name="s")

    @functools.partial(
        pl.kernel,
        mesh=mesh,
        out_type=jax.ShapeDtypeStruct((C, D_MODEL, R), jnp.float32),
        scratch_types=[
            pltpu.VMEM((RBLK,), jnp.int32),
            pltpu.VMEM((RBLK, D_MODEL), jnp.float32),
            pltpu.VMEM((D_MODEL, RBLK), jnp.float32),
            pltpu.SemaphoreType.DMA,
        ],
        compiler_params=pltpu.CompilerParams(
            use_tc_tiling_on_sc=False, needs_layout_passes=False
        ),
    )
    def k(xf_hbm, lut_hbm, out_hbm, idx_v, gat_v, out_v, sem):
        wid = lax.axis_index("s") * NUM_CORES + lax.axis_index("c")
        iota = lax.iota(jnp.int32, LANES)
        scale = jnp.float32(SCALE)

        @pl.loop(0, t_per_w)
        def _(t):
            bid = wid * t_per_w + t
            j = lax.shift_right_logical(bid, 5)
            rb = lax.bitwise_and(bid, nrb - 1)

            pltpu.sync_copy(xf_hbm.at[pl.ds(bid * RBLK, RBLK)], idx_v)
            pltpu.async_copy(lut_hbm.at[idx_v], gat_v, sem).wait()

            @pl.loop(0, RBLK // LANES)
            def _(g):
                rvec = g * LANES + iota

                @plsc.parallel_loop(0, D_MODEL, unroll=8)
                def _(d):
                    v = plsc.load_gather(gat_v, [rvec, jnp.full((LANES,), d, jnp.int32)])
                    out_v[d, pl.ds(g * LANES, LANES)] = v * scale

            pltpu.sync_copy(out_v, out_hbm.at[j, :, pl.ds(rb * RBLK, RBLK)])

    return k(xf, lut)


def kernel(x, lut):
    R, C = x.shape
    xf = jnp.transpose(x.astype(jnp.int32)).reshape(-1)
    out_t = _sc_embed(xf, lut, R, C)
    return jnp.transpose(out_t, (2, 0, 1))


# speedup vs baseline: 2.2444x; 1.4665x over previous
import functools
import math

import jax
import jax.numpy as jnp
from jax import lax
from jax.experimental import pallas as pl
from jax.experimental.pallas import tpu as pltpu
from jax.experimental.pallas import tpu_sc as plsc

D_MODEL = 64
SCALE = math.sqrt(D_MODEL)
NUM_CORES = 2
NUM_SUBCORES = 16
NUM_WORKERS = NUM_CORES * NUM_SUBCORES
LANES = 16
RBLK = 128


@functools.partial(jax.jit, static_argnums=(2, 3))
def _sc_embed(xt, lut, R, C):
    nrb = R // RBLK
    n_blocks = C * nrb
    t_per_w = n_blocks // NUM_WORKERS
    mesh = plsc.VectorSubcoreMesh(core_axis_name="c", subcore_axis_name="s")

    @functools.partial(
        pl.kernel,
        mesh=mesh,
        out_type=jax.ShapeDtypeStruct((C, D_MODEL, R), jnp.float32),
        scratch_types=[
            pltpu.VMEM((RBLK,), jnp.int32),
            pltpu.VMEM((RBLK, D_MODEL), jnp.float32),
            pltpu.VMEM((D_MODEL, RBLK), jnp.float32),
            pltpu.SemaphoreType.DMA,
        ],
        compiler_params=pltpu.CompilerParams(
            use_tc_tiling_on_sc=False, needs_layout_passes=False
        ),
    )
    def k(xt_hbm, lut_hbm, out_hbm, idx_v, gat_v, out_v, sem):
        wid = lax.axis_index("s") * NUM_CORES + lax.axis_index("c")
        iota = lax.iota(jnp.int32, LANES)
        scale = jnp.float32(SCALE)

        @pl.loop(0, t_per_w)
        def _(t):
            bid = wid * t_per_w + t
            j = lax.shift_right_logical(bid, 5)
            rb = lax.bitwise_and(bid, nrb - 1)

            pltpu.sync_copy(xt_hbm.at[j, rb], idx_v)
            pltpu.async_copy(lut_hbm.at[idx_v], gat_v, sem).wait()

            @pl.loop(0, RBLK // LANES)
            def _(rg):
                r0 = rg * LANES

                @plsc.parallel_loop(0, LANES, unroll=4)
                def _(s):
                    rvec = r0 + lax.bitwise_and(iota + s, LANES - 1)
                    for dg in range(D_MODEL // LANES):
                        dvec = dg * LANES + iota
                        v = plsc.load_gather(gat_v, [rvec, dvec])
                        plsc.store_scatter(out_v, [dvec, rvec], v * scale)

            pltpu.sync_copy(out_v, out_hbm.at[j, :, pl.ds(rb * RBLK, RBLK)])

    return k(xt, lut)


def kernel(x, lut):
    R, C = x.shape
    xt = jnp.transpose(x.astype(jnp.int32)).reshape(C, R // RBLK, RBLK)
    out_t = _sc_embed(xt, lut, R, C)
    return jnp.transpose(out_t, (2, 0, 1))


# speedup vs baseline: 3.0423x; 1.3555x over previous
import functools
import math

import jax
import jax.numpy as jnp
from jax import lax
from jax.experimental import pallas as pl
from jax.experimental.pallas import tpu as pltpu
from jax.experimental.pallas import tpu_sc as plsc

D_MODEL = 64
SCALE = math.sqrt(D_MODEL)
NUM_CORES = 2
NUM_SUBCORES = 16
NUM_WORKERS = NUM_CORES * NUM_SUBCORES
LANES = 16
RBLK = 128
NBUF = 4
IBUF = 2 * NBUF


@functools.partial(jax.jit, static_argnums=(2, 3))
def _sc_embed(xt, lut, R, C):
    nrb = R // RBLK
    n_blocks = C * nrb
    t_per_w = n_blocks // NUM_WORKERS
    mesh = plsc.VectorSubcoreMesh(core_axis_name="c", subcore_axis_name="s")

    @functools.partial(
        pl.kernel,
        mesh=mesh,
        out_type=jax.ShapeDtypeStruct((C, D_MODEL, R), jnp.float32),
        scratch_types=(
            [pltpu.VMEM((RBLK,), jnp.int32)] * IBUF
            + [pltpu.VMEM((RBLK, D_MODEL), jnp.float32)] * NBUF
            + [pltpu.VMEM((D_MODEL, RBLK), jnp.float32)] * NBUF
            + [pltpu.SemaphoreType.DMA] * (IBUF + 2 * NBUF)
        ),
        compiler_params=pltpu.CompilerParams(
            use_tc_tiling_on_sc=False, needs_layout_passes=False
        ),
    )
    def k(xt_hbm, lut_hbm, out_hbm, *bufs):
        idxs = bufs[:IBUF]
        gats = bufs[IBUF:IBUF + NBUF]
        outs = bufs[IBUF + NBUF:IBUF + 2 * NBUF]
        isems = bufs[IBUF + 2 * NBUF:2 * IBUF + 2 * NBUF]
        gsems = bufs[2 * IBUF + 2 * NBUF:2 * IBUF + 3 * NBUF]
        osems = bufs[2 * IBUF + 3 * NBUF:]

        wid = lax.axis_index("s") * NUM_CORES + lax.axis_index("c")
        base = wid * t_per_w
        iota = lax.iota(jnp.int32, LANES)
        scale = jnp.float32(SCALE)

        def jrb(t):
            bid = base + t
            return lax.shift_right_logical(bid, 5), lax.bitwise_and(bid, nrb - 1)

        def icopy(t, ib):
            j, rb = jrb(t)
            return pltpu.make_async_copy(xt_hbm.at[j, rb], idxs[ib], isems[ib])

        def gcopy(t, b, ib):
            return pltpu.make_async_copy(
                lut_hbm.at[idxs[ib]], gats[b], gsems[b])

        def ocopy(t, b):
            j, rb = jrb(t)
            return pltpu.make_async_copy(
                outs[b], out_hbm.at[j, :, pl.ds(rb * RBLK, RBLK)], osems[b])

        for t in range(IBUF):
            icopy(t, t).start()
        for t in range(NBUF):
            icopy(t, t).wait()
            gcopy(t, t, t).start()

        @pl.loop(0, t_per_w, step=IBUF)
        def _(g):
            for k in range(IBUF):
                t = g + k
                b = k % NBUF
                gcopy(t, b, k).wait()

                @pl.when(t + IBUF < t_per_w)
                def _():
                    icopy(t + IBUF, k).start()

                @pl.when(t >= NBUF)
                def _():
                    ocopy(t - NBUF, b).wait()

                @pl.loop(0, RBLK // LANES)
                def _(rg):
                    r0 = rg * LANES

                    @plsc.parallel_loop(0, LANES, unroll=4)
                    def _(s):
                        rvec = r0 + lax.bitwise_and(iota + s, LANES - 1)
                        for dg in range(D_MODEL // LANES):
                            dvec = dg * LANES + iota
                            v = plsc.load_gather(gats[b], [rvec, dvec])
                            plsc.store_scatter(outs[b], [dvec, rvec], v * scale)

                ocopy(t, b).start()

                @pl.when(t + NBUF < t_per_w)
                def _():
                    icopy(t + NBUF, (k + NBUF) % IBUF).wait()
                    gcopy(t + NBUF, b, (k + NBUF) % IBUF).start()

        for k in range(NBUF):
            ocopy(t_per_w - NBUF + k, k % NBUF).wait()

    return k(xt, lut)


def kernel(x, lut):
    R, C = x.shape
    xt = jnp.transpose(x.astype(jnp.int32)).reshape(C, R // RBLK, RBLK)
    out_t = _sc_embed(xt, lut, R, C)
    return jnp.transpose(out_t, (2, 0, 1))


# speedup vs baseline: 3.8803x; 1.2754x over previous
import functools
import math

import jax
import jax.numpy as jnp
from jax import lax
from jax.experimental import pallas as pl
from jax.experimental.pallas import tpu as pltpu
from jax.experimental.pallas import tpu_sc as plsc

D_MODEL = 64
SCALE = math.sqrt(D_MODEL)
NUM_CORES = 2
NUM_SUBCORES = 16
NUM_WORKERS = NUM_CORES * NUM_SUBCORES
LANES = 16
RBLK = 128
NBUF = 4
IBUF = 2 * NBUF


@functools.partial(jax.jit, static_argnums=(2, 3))
def _sc_embed(xq, lut, R, C):
    nrb = R // RBLK
    n_blocks = C * nrb
    t_per_w = n_blocks // NUM_WORKERS
    mesh = plsc.VectorSubcoreMesh(core_axis_name="c", subcore_axis_name="s")

    @functools.partial(
        pl.kernel,
        mesh=mesh,
        out_type=jax.ShapeDtypeStruct(
            (C, D_MODEL // 8, R // RBLK, 8, RBLK), jnp.float32),
        scratch_types=(
            [pltpu.VMEM((RBLK,), jnp.int32)] * IBUF
            + [pltpu.VMEM((RBLK, D_MODEL), jnp.float32)] * NBUF
            + [pltpu.VMEM((8, 8, RBLK), jnp.float32)] * NBUF
            + [pltpu.SemaphoreType.DMA] * (IBUF + 2 * NBUF)
        ),
        compiler_params=pltpu.CompilerParams(
            use_tc_tiling_on_sc=False, needs_layout_passes=False
        ),
    )
    def k(xq_hbm, lut_hbm, out_hbm, *bufs):
        idxs = bufs[:IBUF]
        gats = bufs[IBUF:IBUF + NBUF]
        outs = bufs[IBUF + NBUF:IBUF + 2 * NBUF]
        isems = bufs[IBUF + 2 * NBUF:2 * IBUF + 2 * NBUF]
        gsems = bufs[2 * IBUF + 2 * NBUF:2 * IBUF + 3 * NBUF]
        osems = bufs[2 * IBUF + 3 * NBUF:]

        wid = lax.axis_index("s") * NUM_CORES + lax.axis_index("c")
        base = wid * t_per_w
        iota = lax.iota(jnp.int32, LANES)
        scale = jnp.float32(SCALE)

        def jrb(t):
            bid = base + t
            return lax.shift_right_logical(bid, 5), lax.bitwise_and(bid, nrb - 1)

        def icopy(t, ib):
            j, rb = jrb(t)
            return pltpu.make_async_copy(
                xq_hbm.at[lax.shift_right_logical(j, 3), rb,
                          lax.bitwise_and(j, 7)],
                idxs[ib], isems[ib])

        def gcopy(t, b, ib):
            return pltpu.make_async_copy(
                lut_hbm.at[idxs[ib]], gats[b], gsems[b])

        def ocopy(t, b):
            j, rb = jrb(t)
            return pltpu.make_async_copy(
                outs[b], out_hbm.at[j, :, rb], osems[b])

        for t in range(IBUF):
            icopy(t, t).start()
        for t in range(NBUF):
            icopy(t, t).wait()
            gcopy(t, t, t).start()

        @pl.loop(0, t_per_w, step=IBUF)
        def _(g):
            for u in range(IBUF):
                t = g + u
                b = u % NBUF
                gcopy(t, b, u).wait()

                @pl.when(t + IBUF < t_per_w)
                def _():
                    icopy(t + IBUF, u).start()

                @pl.when(t >= NBUF)
                def _():
                    ocopy(t - NBUF, b).wait()

                @pl.loop(0, RBLK // LANES)
                def _(rg):
                    r0 = rg * LANES

                    @plsc.parallel_loop(0, LANES, unroll=4)
                    def _(s):
                        rvec = r0 + lax.bitwise_and(iota + s, LANES - 1)
                        for dg in range(D_MODEL // LANES):
                            dvec = dg * LANES + iota
                            v = plsc.load_gather(gats[b], [rvec, dvec])
                            plsc.store_scatter(
                                outs[b],
                                [lax.shift_right_logical(dvec, 3),
                                 lax.bitwise_and(dvec, 7), rvec],
                                v * scale)

                ocopy(t, b).start()

                @pl.when(t + NBUF < t_per_w)
                def _():
                    icopy(t + NBUF, (u + NBUF) % IBUF).wait()
                    gcopy(t + NBUF, b, (u + NBUF) % IBUF).start()

        for u in range(NBUF):
            ocopy(t_per_w - NBUF + u, u % NBUF).wait()

    return k(xq, lut)


def kernel(x, lut):
    R, C = x.shape
    xq = jnp.transpose(
        jnp.transpose(x.astype(jnp.int32)).reshape(C // 8, 8, R // RBLK, RBLK),
        (0, 2, 1, 3))
    out5 = _sc_embed(xq, lut, R, C)
    return jnp.transpose(out5, (2, 4, 0, 1, 3)).reshape(R, C, D_MODEL)
